# baseline (speedup 1.0000x reference)
import functools

import jax
import jax.numpy as jnp
from jax import lax
from jax.experimental import pallas as pl
from jax.experimental.pallas import tpu as pltpu
from jax.experimental.pallas import tpu_sc as plsc

_B = 64
_H = 2048
_V = 32000
_K = 8
_VB = 1280
_NB = _V // _VB

_NEG_INF = float("-inf")
_IMAX = 2**31 - 1


def _extract_topk(vals, idxs, k):
    outs_v, outs_i = [], []
    for _ in range(k):
        mv = jnp.max(vals, axis=1, keepdims=True)
        mi = jnp.min(jnp.where(vals == mv, idxs, _IMAX), axis=1, keepdims=True)
        outs_v.append(mv)
        outs_i.append(mi)
        vals = jnp.where(idxs == mi, _NEG_INF, vals)
    return jnp.concatenate(outs_v, axis=1), jnp.concatenate(outs_i, axis=1)


def _tc_body(hs_ref, w1_ref, w2_ref, topi_out, scores_out,
             m_ref, s_ref, tv_ref, ti_ref):
    j = pl.program_id(0)

    @pl.when(j == 0)
    def _():
        m_ref[...] = jnp.full((_B, 128), -jnp.inf, jnp.float32)
        s_ref[...] = jnp.zeros((_B, 128), jnp.float32)
        tv_ref[...] = jnp.full((_K, _B, 128), -jnp.inf, jnp.float32)
        ti_ref[...] = jnp.zeros((_K, _B, 128), jnp.int32)

    halves = []
    for h, w_ref in enumerate((w1_ref, w2_ref)):
        halves.append(lax.dot_general(
            hs_ref[...], w_ref[...],
            (((1,), (1,)), ((), ())),
            preferred_element_type=jnp.float32,
        ))

    tv = [tv_ref[r] for r in range(_K)]
    ti = [ti_ref[r] for r in range(_K)]
    lane = lax.broadcasted_iota(jnp.int32, (_B, 128), 1)
    hw = _VB // 2
    for h, block in enumerate(halves):
        for s in range(hw // 128):
            new_v = block[:, s * 128:(s + 1) * 128]
            new_i = (j * _VB + h * hw + s * 128) + lane
            for r in range(_K):
                gt = new_v > tv[r]
                keep_v, keep_i = tv[r], ti[r]
                tv[r] = jnp.where(gt, new_v, keep_v)
                ti[r] = jnp.where(gt, new_i, keep_i)
                new_v = jnp.where(gt, keep_v, new_v)
                new_i = jnp.where(gt, keep_i, new_i)
    for r in range(_K):
        tv_ref[r] = tv[r]
        ti_ref[r] = ti[r]

    m_prev = m_ref[:, 0:1]
    new_m = jnp.max(tv[0], axis=1, keepdims=True)
    bs = (jnp.sum(jnp.exp(halves[0] - new_m), axis=1, keepdims=True)
          + jnp.sum(jnp.exp(halves[1] - new_m), axis=1, keepdims=True))
    s_run = s_ref[:, 0:1] * jnp.exp(m_prev - new_m) + bs
    m_ref[:, 0:1] = new_m
    s_ref[:, 0:1] = s_run

    @pl.when(j == _NB - 1)
    def _():
        cv = jnp.concatenate(tv, axis=1)
        ci = jnp.concatenate(ti, axis=1)
        nv, ni = _extract_topk(cv, ci, _K)
        lse = new_m + jnp.log(s_run)
        scores_out[...] = nv - lse
        topi_out[...] = ni


def _tc_topk(hidden_states, w_lm):
    hb = _VB // 2
    return pl.pallas_call(
        _tc_body,
        grid=(_NB,),
        in_specs=[
            pl.BlockSpec((_B, _H), lambda j: (0, 0)),
            pl.BlockSpec((hb, _H), lambda j: (2 * j, 0)),
            pl.BlockSpec((hb, _H), lambda j: (2 * j + 1, 0)),
        ],
        out_specs=[
            pl.BlockSpec((_B, _K), lambda j: (0, 0)),
            pl.BlockSpec((_B, _K), lambda j: (0, 0)),
        ],
        out_shape=[
            jax.ShapeDtypeStruct((_B, _K), jnp.int32),
            jax.ShapeDtypeStruct((_B, _K), jnp.float32),
        ],
        scratch_shapes=[
            pltpu.VMEM((_B, 128), jnp.float32),
            pltpu.VMEM((_B, 128), jnp.float32),
            pltpu.VMEM((_K, _B, 128), jnp.float32),
            pltpu.VMEM((_K, _B, 128), jnp.int32),
        ],
        compiler_params=pltpu.CompilerParams(
            dimension_semantics=("arbitrary",)),
    )(hidden_states, w_lm, w_lm)


_N_IDX = _B * _K


_PER_TILE = _N_IDX // 16


def _sc_remap_body(d2t_hbm, idx_hbm, out_hbm, idx_v, val_v, out_v, sem):
    base = lax.axis_index("s") * _PER_TILE
    pltpu.sync_copy(idx_hbm.at[pl.ds(base, _PER_TILE)], idx_v)
    pltpu.async_copy(d2t_hbm.at[idx_v], val_v, sem).wait()
    for c in range(_PER_TILE // 16):
        sl = pl.ds(c * 16, 16)
        out_v[sl] = idx_v[sl] + val_v[sl]
    pltpu.sync_copy(out_v, out_hbm.at[pl.ds(base, _PER_TILE)])


@functools.cache
def _sc_remap():
    mesh = plsc.VectorSubcoreMesh(
        core_axis_name="c", subcore_axis_name="s", num_cores=1)
    return pl.kernel(
        _sc_remap_body,
        mesh=mesh,
        out_type=jax.ShapeDtypeStruct((_N_IDX,), jnp.int32),
        scratch_types=[
            pltpu.VMEM((_PER_TILE,), jnp.int32),
            pltpu.VMEM((_PER_TILE,), jnp.int32),
            pltpu.VMEM((_PER_TILE,), jnp.int32),
            pltpu.SemaphoreType.DMA,
        ],
    )


def kernel(hidden_states, d2t, W_lm):
    topi, scores = _tc_topk(hidden_states, W_lm)
    mapped = _sc_remap()(d2t, topi.reshape(-1)).reshape(_B, _K)
    return mapped, scores


# speedup vs baseline: 1.0108x; 1.0108x over previous
import functools

import jax
import jax.numpy as jnp
from jax import lax
from jax.experimental import pallas as pl
from jax.experimental.pallas import tpu as pltpu
from jax.experimental.pallas import tpu_sc as plsc

_B = 64
_H = 2048
_V = 32000
_K = 8
_VB = 1280
_NB = _V // _VB

_NEG_INF = float("-inf")
_IMAX = 2**31 - 1


def _extract_topk(vals, idxs, k):
    outs_v, outs_i = [], []
    for _ in range(k):
        mv = jnp.max(vals, axis=1, keepdims=True)
        mi = jnp.min(jnp.where(vals == mv, idxs, _IMAX), axis=1, keepdims=True)
        outs_v.append(mv)
        outs_i.append(mi)
        vals = jnp.where(idxs == mi, _NEG_INF, vals)
    return jnp.concatenate(outs_v, axis=1), jnp.concatenate(outs_i, axis=1)


def _tc_body(hs_ref, w_ref, topi_out, scores_out, m_ref, s_ref, tv_ref, ti_ref):
    j = pl.program_id(0)

    @pl.when(j == 0)
    def _():
        m_ref[...] = jnp.full((_B, 128), -jnp.inf, jnp.float32)
        s_ref[...] = jnp.zeros((_B, 128), jnp.float32)
        tv_ref[...] = jnp.full((_K, _B, 128), -jnp.inf, jnp.float32)
        ti_ref[...] = jnp.zeros((_K, _B, 128), jnp.int32)

    block = lax.dot_general(
        hs_ref[...], w_ref[...],
        (((1,), (1,)), ((), ())),
        preferred_element_type=jnp.float32,
    )

    tv = [tv_ref[r] for r in range(_K)]
    ti = [ti_ref[r] for r in range(_K)]
    lane = lax.broadcasted_iota(jnp.int32, (_B, 128), 1)
    for s in range(_VB // 128):
        new_v = block[:, s * 128:(s + 1) * 128]
        new_i = (j * _VB + s * 128) + lane
        for r in range(_K):
            gt = new_v > tv[r]
            keep_v, keep_i = tv[r], ti[r]
            tv[r] = jnp.where(gt, new_v, keep_v)
            ti[r] = jnp.where(gt, new_i, keep_i)
            new_v = jnp.where(gt, keep_v, new_v)
            new_i = jnp.where(gt, keep_i, new_i)
    for r in range(_K):
        tv_ref[r] = tv[r]
        ti_ref[r] = ti[r]

    m_prev = m_ref[:, 0:1]
    new_m = jnp.max(tv[0], axis=1, keepdims=True)
    bs = jnp.sum(jnp.exp(block - new_m), axis=1, keepdims=True)
    s_run = s_ref[:, 0:1] * jnp.exp(m_prev - new_m) + bs
    m_ref[:, 0:1] = new_m
    s_ref[:, 0:1] = s_run

    @pl.when(j == _NB - 1)
    def _():
        cv = jnp.concatenate(tv, axis=1)
        ci = jnp.concatenate(ti, axis=1)
        nv, ni = _extract_topk(cv, ci, _K)
        lse = new_m + jnp.log(s_run)
        scores_out[...] = nv - lse
        topi_out[...] = ni


def _tc_topk(hidden_states, w_lm):
    return pl.pallas_call(
        _tc_body,
        grid=(_NB,),
        in_specs=[
            pl.BlockSpec((_B, _H), lambda j: (0, 0)),
            pl.BlockSpec((_VB, _H), lambda j: (j, 0)),
        ],
        out_specs=[
            pl.BlockSpec((_B, _K), lambda j: (0, 0)),
            pl.BlockSpec((_B, _K), lambda j: (0, 0)),
        ],
        out_shape=[
            jax.ShapeDtypeStruct((_B, _K), jnp.int32),
            jax.ShapeDtypeStruct((_B, _K), jnp.float32),
        ],
        scratch_shapes=[
            pltpu.VMEM((_B, 128), jnp.float32),
            pltpu.VMEM((_B, 128), jnp.float32),
            pltpu.VMEM((_K, _B, 128), jnp.float32),
            pltpu.VMEM((_K, _B, 128), jnp.int32),
        ],
        compiler_params=pltpu.CompilerParams(
            dimension_semantics=("arbitrary",)),
    )(hidden_states, w_lm)


_N_IDX = _B * _K


_PER_TILE = _N_IDX // 16


def _sc_remap_body(d2t_hbm, idx_hbm, out_hbm, idx_v, val_v, out_v, sem):
    base = lax.axis_index("s") * _PER_TILE
    pltpu.sync_copy(idx_hbm.at[pl.ds(base, _PER_TILE)], idx_v)
    pltpu.async_copy(d2t_hbm.at[idx_v], val_v, sem).wait()
    for c in range(_PER_TILE // 16):
        sl = pl.ds(c * 16, 16)
        out_v[sl] = idx_v[sl] + val_v[sl]
    pltpu.sync_copy(out_v, out_hbm.at[pl.ds(base, _PER_TILE)])


@functools.cache
def _sc_remap():
    mesh = plsc.VectorSubcoreMesh(
        core_axis_name="c", subcore_axis_name="s", num_cores=1)
    return pl.kernel(
        _sc_remap_body,
        mesh=mesh,
        out_type=jax.ShapeDtypeStruct((_N_IDX,), jnp.int32),
        scratch_types=[
            pltpu.VMEM((_PER_TILE,), jnp.int32),
            pltpu.VMEM((_PER_TILE,), jnp.int32),
            pltpu.VMEM((_PER_TILE,), jnp.int32),
            pltpu.SemaphoreType.DMA,
        ],
    )


def kernel(hidden_states, d2t, W_lm):
    topi, scores = _tc_topk(hidden_states, W_lm)
    mapped = _sc_remap()(d2t, topi.reshape(-1)).reshape(_B, _K)
    return mapped, scores
